# baseline (speedup 1.0000x reference)
import jax
import jax.numpy as jnp
from jax import lax
from jax.experimental import pallas as pl
from jax.experimental.pallas import tpu as pltpu
from jax.experimental.pallas import tpu_sc as plsc

_NC = 2
_NS = 16
_NW = _NC * _NS

_B, _N, _S, _D = 4, 16, 2048, 128
_ROW = _S * _D
_PIECE = 32768
_ROWP = _ROW // _PIECE
_NJ = 60
_NQ = _B * _NJ * _ROWP
_QPW = _NQ // _NW


def _src_offset(q):
    r, p = q // _ROWP, q % _ROWP
    b, j = r // _NJ, r % _NJ
    h, m = j // 30, j % 30
    head = jnp.where(h == 0, m, 3 * (m % 3) + m // 3)
    t, g = (m - 9) % 3, (m - 9) // 3
    pair = jnp.where(h == 0, 6 + t, 2 + 3 * t)
    tail = jnp.where(t < 2, pair, g + 9)
    idx = jnp.where(m < 9, head, tail)
    return ((b * _N + idx) * _ROWP + p) * _PIECE


def _body(x_hbm, out_hbm, buf0, buf1, buf2, sg0, sg1, sg2, ss0, ss1, ss2):
    c = lax.axis_index("c")
    s = lax.axis_index("s")
    w = s * _NC + c
    bufs, sgs, sss = (buf0, buf1, buf2), (sg0, sg1, sg2), (ss0, ss1, ss2)

    def start_gather(k, b):
        off = pl.multiple_of(_src_offset(w * _QPW + k), _PIECE)
        pltpu.async_copy(x_hbm.at[pl.ds(off, _PIECE)], bufs[b], sgs[b])

    def wait_gather(b):
        pltpu.make_async_copy(x_hbm.at[pl.ds(0, _PIECE)], bufs[b],
                              sgs[b]).wait()

    def start_store(k, b):
        off = pl.multiple_of((w * _QPW + k) * _PIECE, _PIECE)
        pltpu.async_copy(bufs[b], out_hbm.at[pl.ds(off, _PIECE)], sss[b])

    def wait_store(b):
        pltpu.make_async_copy(bufs[b], out_hbm.at[pl.ds(0, _PIECE)],
                              sss[b]).wait()

    start_gather(0, 0)
    start_gather(1, 1)
    wait_gather(0)
    start_store(0, 0)
    start_gather(2, 2)

    @pl.loop(0, (_QPW - 3) // 3)
    def _(t):
        for sl in (1, 2, 3):
            k = 3 * t + sl
            b, bn = sl % 3, (sl + 2) % 3
            wait_gather(b)
            start_store(k, b)
            wait_store(bn)
            start_gather(k + 2, bn)

    wait_gather(1)
    start_store(_QPW - 2, 1)
    wait_store(0)
    wait_gather(2)
    start_store(_QPW - 1, 2)
    wait_store(1)
    wait_store(2)


@jax.jit
def kernel(x):
    b, n, s, d = x.shape
    x1 = x.reshape(-1)
    mesh = plsc.VectorSubcoreMesh(core_axis_name="c", subcore_axis_name="s")
    out = pl.kernel(
        _body,
        out_type=jax.ShapeDtypeStruct((_NQ * _PIECE,), jnp.float32),
        mesh=mesh,
        scratch_types=(
            [pltpu.VMEM((_PIECE,), jnp.float32)] * 3
            + [pltpu.SemaphoreType.DMA] * 6
        ),
    )(x1)
    return out.reshape(b, _NJ // 3, 3, s, d)


# speedup vs baseline: 1.0721x; 1.0721x over previous
import jax
import jax.numpy as jnp
from jax import lax
from jax.experimental import pallas as pl
from jax.experimental.pallas import tpu as pltpu
from jax.experimental.pallas import tpu_sc as plsc

_NC = 2
_NS = 16
_NW = _NC * _NS

_B, _N, _S, _D = 4, 16, 2048, 128
_ROW = _S * _D
_PIECE = 32768
_ROWP = _ROW // _PIECE
_NJ = 60
_NQ = _B * _NJ * _ROWP
_QPW = _NQ // _NW


def _src_offset(q):
    r, p = q // _ROWP, q % _ROWP
    b, j = r // _NJ, r % _NJ
    h, m = j // 30, j % 30
    head = jnp.where(h == 0, m, 3 * (m % 3) + m // 3)
    t, g = (m - 9) % 3, (m - 9) // 3
    pair = jnp.where(h == 0, 6 + t, 2 + 3 * t)
    tail = jnp.where(t < 2, pair, g + 9)
    idx = jnp.where(m < 9, head, tail)
    return ((b * _N + idx) * _ROWP + p) * _PIECE


def _body(x_hbm, out_hbm, buf0, buf1, sg0, sg1, ss0, ss1):
    c = lax.axis_index("c")
    s = lax.axis_index("s")
    w = s * _NC + c
    bufs = (buf0.at[s], buf1.at[s])
    sgs, sss = (sg0, sg1), (ss0, ss1)

    def start_gather(k, b):
        off = pl.multiple_of(_src_offset(w * _QPW + k), _PIECE)
        pltpu.async_copy(x_hbm.at[pl.ds(off, _PIECE)], bufs[b], sgs[b])

    def wait_gather(b):
        pltpu.make_async_copy(x_hbm.at[pl.ds(0, _PIECE)], bufs[b],
                              sgs[b]).wait()

    def start_store(k, b):
        off = pl.multiple_of((w * _QPW + k) * _PIECE, _PIECE)
        pltpu.async_copy(bufs[b], out_hbm.at[pl.ds(off, _PIECE)], sss[b])

    def wait_store(b):
        pltpu.make_async_copy(bufs[b], out_hbm.at[pl.ds(0, _PIECE)],
                              sss[b]).wait()

    start_gather(0, 0)
    wait_gather(0)
    start_gather(1, 1)
    start_store(0, 0)

    @pl.loop(1, _QPW // 2)
    def _(t):
        k1 = 2 * t - 1
        wait_gather(1)
        wait_store(0)
        start_gather(k1 + 1, 0)
        start_store(k1, 1)
        k2 = 2 * t
        wait_gather(0)
        wait_store(1)
        start_gather(k2 + 1, 1)
        start_store(k2, 0)

    wait_gather(1)
    wait_store(0)
    start_store(_QPW - 1, 1)
    wait_store(1)


@jax.jit
def kernel(x):
    b, n, s, d = x.shape
    x1 = x.reshape(-1)
    mesh = plsc.VectorSubcoreMesh(core_axis_name="c", subcore_axis_name="s")
    out = pl.kernel(
        _body,
        out_type=jax.ShapeDtypeStruct((_NQ * _PIECE,), jnp.float32),
        mesh=mesh,
        scratch_types=[
            pltpu.VMEM_SHARED((_NS, _PIECE), jnp.float32),
            pltpu.VMEM_SHARED((_NS, _PIECE), jnp.float32),
            pltpu.SemaphoreType.DMA,
            pltpu.SemaphoreType.DMA,
            pltpu.SemaphoreType.DMA,
            pltpu.SemaphoreType.DMA,
        ],
    )(x1)
    return out.reshape(b, _NJ // 3, 3, s, d)
